# Initial kernel scaffold; baseline (speedup 1.0000x reference)
import jax
import jax.numpy as jnp
from jax.experimental import pallas as pl


def kernel(x, edge_index, edge_weight, W1, b1, W2, b2):
    raise NotImplementedError("write your pallas kernel here")


# speedup vs baseline: 15.5580x; 15.5580x over previous
import functools

import jax
import jax.numpy as jnp
from jax import lax
from jax.experimental import pallas as pl
from jax.experimental.pallas import tpu as pltpu
from jax.experimental.pallas import tpu_sc as plsc

N = 10000
E = 320000
DIN = 128
DH = 200
DHP = 256
DOP = 16

NC, NS = 2, 16
NW = NC * NS
CH = 128
EPT_CH = 80
EPT = EPT_CH * CH
EP = NW * EPT
NP = 10240
RPT = NP // NS
CPP = 16

_MESH = plsc.VectorSubcoreMesh(core_axis_name="c", subcore_axis_name="s",
                               num_cores=NC, num_subcores=NS)


def _wid_base(ncols):
    cid = lax.axis_index("c")
    sid = lax.axis_index("s")
    wid = cid * NS + sid
    return cid, sid, wid


def _deg_body(dst_hbm, ew_hbm, z_hbm, out_hbm, acc, idxd, ewv, rows):
    cid, sid, rb = _wid_base(DOP)
    pltpu.sync_copy(z_hbm.at[pl.ds(sid * RPT, RPT)], acc.at[pl.ds(sid * RPT, RPT)])
    pltpu.sync_copy(dst_hbm.at[rb], idxd)
    pltpu.sync_copy(ew_hbm.at[rb], ewv)
    plsc.subcore_barrier()

    @pl.loop(0, EPT_CH)
    def _chunk(c):
        @pl.loop(0, CH // 16)
        def _grp(k):
            w16 = ewv[c, pl.ds(k * 16, 16)]
            for i in range(16):
                rows[k * 16 + i, :] = w16[i] * jnp.ones((16,), jnp.float32)
        pltpu.sync_copy(rows, acc.at[idxd.at[c]], add=True)

    plsc.subcore_barrier()
    pltpu.sync_copy(acc.at[pl.ds(sid * RPT, RPT)],
                    out_hbm.at[cid, pl.ds(sid * RPT, RPT)])


_deg_kernel = pl.kernel(
    _deg_body,
    out_type=jax.ShapeDtypeStruct((NC, NP, DOP), jnp.float32),
    mesh=_MESH,
    compiler_params=pltpu.CompilerParams(use_tc_tiling_on_sc=False),
    scratch_types=[
        pltpu.VMEM_SHARED((NP, DOP), jnp.float32),
        pltpu.VMEM((EPT_CH, CH), jnp.int32),
        pltpu.VMEM((EPT_CH, CH), jnp.float32),
        pltpu.VMEM((CH, DOP), jnp.float32),
    ],
)


def _agg_body(ncols, feat_hbm, src_hbm, dst_hbm, ew_hbm, z_hbm, out_hbm,
              acc, idxs, idxd, ewv, rows0, rows1, sem0, sem1):
    cid, sid, rb = _wid_base(ncols)
    pltpu.sync_copy(z_hbm.at[pl.ds(sid * RPT, RPT)], acc.at[pl.ds(sid * RPT, RPT)])
    plsc.subcore_barrier()

    nvec = ncols // 16
    bufs = ((rows0, sem0), (rows1, sem1))

    def process(c, rows, sem):
        pltpu.make_async_copy(feat_hbm.at[idxs.at[c]], rows, sem).wait()

        @pl.loop(0, CH // 16)
        def _grp(k):
            w16 = ewv[c, pl.ds(k * 16, 16)]
            for i in range(16):
                w = w16[i]
                for j in range(nvec):
                    sl = pl.ds(j * 16, 16)
                    rows[k * 16 + i, sl] = rows[k * 16 + i, sl] * w

        pltpu.sync_copy(rows, acc.at[idxd.at[c]], add=True)

        @pl.when(c + 2 < CPP)
        def _():
            pltpu.async_copy(feat_hbm.at[idxs.at[c + 2]], rows, sem)

    for ph in range(EPT_CH // CPP):
        pltpu.sync_copy(src_hbm.at[rb, pl.ds(ph * CPP, CPP)], idxs)
        pltpu.sync_copy(dst_hbm.at[rb, pl.ds(ph * CPP, CPP)], idxd)
        pltpu.sync_copy(ew_hbm.at[rb, pl.ds(ph * CPP, CPP)], ewv)
        pltpu.async_copy(feat_hbm.at[idxs.at[0]], rows0, sem0)
        pltpu.async_copy(feat_hbm.at[idxs.at[1]], rows1, sem1)

        @pl.loop(0, CPP // 2)
        def _pair(p):
            for b, (rows, sem) in enumerate(bufs):
                process(2 * p + b, rows, sem)

    plsc.subcore_barrier()
    pltpu.sync_copy(acc.at[pl.ds(sid * RPT, RPT)],
                    out_hbm.at[cid, pl.ds(sid * RPT, RPT)])


def _make_agg_kernel(ncols, tc_tiling=True):
    return pl.kernel(
        functools.partial(_agg_body, ncols),
        out_type=jax.ShapeDtypeStruct((NC, NP, ncols), jnp.float32),
        mesh=_MESH,
        compiler_params=pltpu.CompilerParams(use_tc_tiling_on_sc=tc_tiling),
        scratch_types=[
            pltpu.VMEM_SHARED((NP, ncols), jnp.float32),
            pltpu.VMEM((CPP, CH), jnp.int32),
            pltpu.VMEM((CPP, CH), jnp.int32),
            pltpu.VMEM((CPP, CH), jnp.float32),
            pltpu.VMEM((CH, ncols), jnp.float32),
            pltpu.VMEM((CH, ncols), jnp.float32),
            pltpu.SemaphoreType.DMA,
            pltpu.SemaphoreType.DMA,
        ],
    )


_agg128 = _make_agg_kernel(DIN)
_agg16 = _make_agg_kernel(DOP, tc_tiling=False)


def _scale_body(deg0_ref, deg1_ref, x_ref, dinv_ref, xs_ref):
    d = deg0_ref[...] + deg1_ref[...] + 1.0
    dv = lax.rsqrt(d)
    dinv_ref[...] = dv
    xs_ref[...] = x_ref[...] * dv


_RB = 1000

_scale_kernel = pl.pallas_call(
    _scale_body,
    grid=(N // _RB,),
    in_specs=[
        pl.BlockSpec((_RB, 1), lambda i: (i, 0)),
        pl.BlockSpec((_RB, 1), lambda i: (i, 0)),
        pl.BlockSpec((_RB, DIN), lambda i: (i, 0)),
    ],
    out_specs=[
        pl.BlockSpec((_RB, 1), lambda i: (i, 0)),
        pl.BlockSpec((_RB, DIN), lambda i: (i, 0)),
    ],
    out_shape=[
        jax.ShapeDtypeStruct((N, 1), jnp.float32),
        jax.ShapeDtypeStruct((N, DIN), jnp.float32),
    ],
)


def _mlp_body(p0_ref, p1_ref, xs_ref, dinv_ref, w1_ref, b1_ref, w2_ref, g_ref):
    t1 = (p0_ref[...] + p1_ref[...] + xs_ref[...]) * dinv_ref[...]
    h = jnp.dot(t1, w1_ref[...], preferred_element_type=jnp.float32) + b1_ref[...]
    h = jnp.maximum(h, 0.0)
    g = jnp.dot(h, w2_ref[...], preferred_element_type=jnp.float32)
    g_ref[...] = g * dinv_ref[...]


_mlp_kernel = pl.pallas_call(
    _mlp_body,
    grid=(N // _RB,),
    in_specs=[
        pl.BlockSpec((_RB, DIN), lambda i: (i, 0)),
        pl.BlockSpec((_RB, DIN), lambda i: (i, 0)),
        pl.BlockSpec((_RB, DIN), lambda i: (i, 0)),
        pl.BlockSpec((_RB, 1), lambda i: (i, 0)),
        pl.BlockSpec((DIN, DHP), lambda i: (0, 0)),
        pl.BlockSpec((1, DHP), lambda i: (0, 0)),
        pl.BlockSpec((DHP, DOP), lambda i: (0, 0)),
    ],
    out_specs=pl.BlockSpec((_RB, DOP), lambda i: (i, 0)),
    out_shape=jax.ShapeDtypeStruct((N, DOP), jnp.float32),
)


def _final_body(p0_ref, p1_ref, g_ref, dinv_ref, b2_ref, out_ref):
    s = (p0_ref[...][:, :8] + p1_ref[...][:, :8] + g_ref[...][:, :8]) * dinv_ref[...]
    out_ref[...] = s + b2_ref[...]


_final_kernel = pl.pallas_call(
    _final_body,
    grid=(N // _RB,),
    in_specs=[
        pl.BlockSpec((_RB, DOP), lambda i: (i, 0)),
        pl.BlockSpec((_RB, DOP), lambda i: (i, 0)),
        pl.BlockSpec((_RB, DOP), lambda i: (i, 0)),
        pl.BlockSpec((_RB, 1), lambda i: (i, 0)),
        pl.BlockSpec((1, 8), lambda i: (0, 0)),
    ],
    out_specs=pl.BlockSpec((_RB, 8), lambda i: (i, 0)),
    out_shape=jax.ShapeDtypeStruct((N, 8), jnp.float32),
)


def kernel(x, edge_index, edge_weight, W1, b1, W2, b2):
    pad = EP - E
    src2d = jnp.concatenate(
        [edge_index[0], jnp.zeros((pad,), jnp.int32)]).reshape(NW, EPT_CH, CH)
    dst2d = jnp.concatenate(
        [edge_index[1], jnp.zeros((pad,), jnp.int32)]).reshape(NW, EPT_CH, CH)
    ew2d = jnp.concatenate(
        [edge_weight, jnp.zeros((pad,), jnp.float32)]).reshape(NW, EPT_CH, CH)

    w1p = jnp.pad(W1, ((0, 0), (0, DHP - DH)))
    b1p = jnp.pad(b1, (0, DHP - DH)).reshape(1, DHP)
    w2p = jnp.pad(W2, ((0, DHP - DH), (0, DOP - 8)))
    b2r = b2.reshape(1, 8)

    z128 = jnp.zeros((NP, DIN), jnp.float32)
    z16 = jnp.zeros((NP, DOP), jnp.float32)

    degp = _deg_kernel(dst2d, ew2d, z16)
    deg0 = degp[0, :N, 0:1]
    deg1 = degp[1, :N, 0:1]

    dinv, xs = _scale_kernel(deg0, deg1, x)

    t1p = _agg128(xs, src2d, dst2d, ew2d, z128)

    g = _mlp_kernel(t1p[0, :N], t1p[1, :N], xs, dinv, w1p, b1p, w2p)

    t2p = _agg16(g, src2d, dst2d, ew2d, z16)

    return _final_kernel(t2p[0, :N], t2p[1, :N], g, dinv, b2r)
